# baseline (speedup 1.0000x reference)
import jax
import jax.numpy as jnp
from jax import lax
from jax.experimental import pallas as pl
from jax.experimental.pallas import tpu as pltpu
from jax.experimental.pallas import tpu_sc as plsc

N_USERS = 25000
N_ITEMS = 25000
N = N_USERS + N_ITEMS
E = 800000
D = 64

NC = 2
NS = 16
NW = NC * NS
HALF = N // NC
RPC = 25088
RPT = RPC // NS
DUMP = HALF
PAD = RPC - HALF
K = 80
SUP = 10
KS = K * SUP
EPS = 25600
EPAD = NW * EPS
NSUP_IN = EPS // KS
CAPB = EPS // K


def _route_body(colf_hbm, rowf_hbm, valf_hbm,
                col2_hbm, lr2_hbm, val2_hbm, cnt_hbm,
                colv, rowv, valv, oc0, ol0, ov0, oc1, ol1, ov1, cntv,
                sgi0, sgi1, sgi2):
    c = lax.axis_index("c")
    s = lax.axis_index("s")
    w = c * NS + s
    e0 = w * EPS
    ocs = (oc0, oc1)
    ols = (ol0, ol1)
    ovs = (ov0, ov1)
    iota = lax.broadcasted_iota(jnp.int32, (16,), 0)

    def flush(cc, blk):
        for r in range(SUP):
            pltpu.sync_copy(ocs[cc].at[pl.ds(r * K, K)],
                            col2_hbm.at[cc, w, blk * SUP + r])
            pltpu.sync_copy(ols[cc].at[pl.ds(r * K, K)],
                            lr2_hbm.at[cc, w, blk * SUP + r])
            pltpu.sync_copy(ovs[cc].at[pl.ds(r * K, K)],
                            val2_hbm.at[cc, w, blk * SUP + r])

    def sup_body(i, carry):
        di0 = pltpu.async_copy(colf_hbm.at[pl.ds(e0 + i * KS, KS)], colv,
                               sgi0)
        di1 = pltpu.async_copy(rowf_hbm.at[pl.ds(e0 + i * KS, KS)], rowv,
                               sgi1)
        di2 = pltpu.async_copy(valf_hbm.at[pl.ds(e0 + i * KS, KS)], valv,
                               sgi2)
        di0.wait()
        di1.wait()
        di2.wait()

        def grp(g, carry2):
            ptrs = [carry2[0], carry2[2]]
            blks = [carry2[1], carry2[3]]
            sl = pl.ds(g * 16, 16)
            c16 = colv[sl]
            r16 = rowv[sl]
            v16 = valv[sl]
            for cc in range(NC):
                ptr = ptrs[cc]
                blk = blks[cc]
                lr = r16 - cc * HALF
                m = (lr >= 0) & (lr < HALF)
                cnt = plsc.all_reduce_population_count(m)[0]
                plsc.store_compressed(ocs[cc].at[pl.ds(ptr, 16)], c16, mask=m)
                plsc.store_compressed(ols[cc].at[pl.ds(ptr, 16)], lr, mask=m)
                plsc.store_compressed(ovs[cc].at[pl.ds(ptr, 16)], v16, mask=m)
                pnew = ptr + cnt
                full = pnew >= KS

                @pl.when(full)
                def _():
                    flush(cc, blk)
                    for t in range(K // 16):
                        dst = pl.ds(t * 16, 16)
                        ssl = pl.ds(KS + t * 16, 16)
                        ocs[cc][dst] = ocs[cc][ssl]
                        ols[cc][dst] = ols[cc][ssl]
                        ovs[cc][dst] = ovs[cc][ssl]

                ptrs[cc] = jnp.where(full, pnew - KS, pnew)
                blks[cc] = jnp.where(full, blk + 1, blk)
            return (ptrs[0], blks[0], ptrs[1], blks[1])

        return lax.fori_loop(0, KS // 16, grp, carry)

    z = jnp.int32(0)
    p0, b0, p1, b1 = lax.fori_loop(0, NSUP_IN, sup_body, (z, z, z, z))

    nsups = []
    for cc, (ptr, blk) in enumerate(((p0, b0), (p1, b1))):
        def pad_grp(t, carry2):
            ofs = ptr + t * 16

            @pl.when(ofs < KS)
            def _():
                ocs[cc][pl.ds(ofs, 16)] = iota * 0
                ols[cc][pl.ds(ofs, 16)] = iota * 0 + DUMP
                ovs[cc][pl.ds(ofs, 16)] = (iota * 0).astype(jnp.float32)

            return carry2

        lax.fori_loop(0, KS // 16, pad_grp, 0)

        @pl.when(ptr > 0)
        def _():
            flush(cc, blk)

        nsups.append(blk + (ptr > 0).astype(jnp.int32))

    cntv[...] = jnp.where(iota == 0, nsups[0],
                          jnp.where(iota == 1, nsups[1], 0))
    pltpu.sync_copy(cntv, cnt_hbm.at[w])


_route = pl.kernel(
    _route_body,
    out_type=[
        jax.ShapeDtypeStruct((NC, NW, CAPB, K), jnp.int32),
        jax.ShapeDtypeStruct((NC, NW, CAPB, K), jnp.int32),
        jax.ShapeDtypeStruct((NC, NW, CAPB, K), jnp.float32),
        jax.ShapeDtypeStruct((NW, 16), jnp.int32),
    ],
    mesh=plsc.VectorSubcoreMesh(core_axis_name="c", subcore_axis_name="s"),
    compiler_params=pltpu.CompilerParams(use_tc_tiling_on_sc=False,
                                         needs_layout_passes=False),
    scratch_types=[
        pltpu.VMEM((KS,), jnp.int32),
        pltpu.VMEM((KS,), jnp.int32),
        pltpu.VMEM((KS,), jnp.float32),
        pltpu.VMEM(((SUP + 1) * K,), jnp.int32),
        pltpu.VMEM(((SUP + 1) * K,), jnp.int32),
        pltpu.VMEM(((SUP + 1) * K,), jnp.float32),
        pltpu.VMEM(((SUP + 1) * K,), jnp.int32),
        pltpu.VMEM(((SUP + 1) * K,), jnp.int32),
        pltpu.VMEM(((SUP + 1) * K,), jnp.float32),
        pltpu.VMEM((16,), jnp.int32),
        pltpu.SemaphoreType.DMA,
        pltpu.SemaphoreType.DMA,
        pltpu.SemaphoreType.DMA,
    ],
)


def _spmm_body(ego_hbm, col2_hbm, lr2_hbm, val2_hbm, cnt_hbm, zeros_hbm,
               out_hbm,
               colv, lrm, vlm, cntb, rb0, rb1, rb2, sb0, sb1, acc,
               sg0, sg1, sg2, ss0, ss1):
    c = lax.axis_index("c")
    s = lax.axis_index("s")
    rbufs = (rb0, rb1, rb2)
    sbufs = (sb0, sb1)
    sgs = (sg0, sg1, sg2)
    sss = (ss0, ss1)

    pltpu.sync_copy(zeros_hbm, acc.at[pl.ds(s * RPT, RPT)])
    plsc.subcore_barrier()

    def mul_pass(rbuf, sbuf, j):
        @plsc.parallel_loop(0, K // 16, unroll=2)
        def group(g):
            v16 = vlm[j, pl.ds(g * 16, 16)]
            for e in range(16):
                k = g * 16 + e
                ve = jnp.take(v16, jnp.full((16,), e, jnp.int32))
                for d_ in range(D // 16):
                    sl = pl.ds(d_ * 16, 16)
                    sbuf[k, sl] = rbuf[k, sl] * ve

    def seg_body(sc_i, p, carry):
        base = pl.ds(sc_i * SUP, SUP)
        di0 = pltpu.async_copy(col2_hbm.at[c, p, base], colv, sg0)
        di1 = pltpu.async_copy(lr2_hbm.at[c, p, base], lrm, sg1)
        di2 = pltpu.async_copy(val2_hbm.at[c, p, base], vlm, sg2)
        di0.wait()
        di1.wait()
        di2.wait()

        dg = [None, None, None]
        dsc = [None, None]
        for jb in range(3):
            dg[jb] = pltpu.async_copy(ego_hbm.at[colv.at[jb]], rbufs[jb],
                                      sgs[jb])
        for j in range(SUP):
            br = j % 3
            bs = j % 2
            dg[br].wait()
            if j >= 2:
                dsc[bs].wait()
            mul_pass(rbufs[br], sbufs[bs], j)
            dsc[bs] = pltpu.async_copy(sbufs[bs], acc.at[lrm.at[j]],
                                       sss[bs], add=True)
            if j + 3 < SUP:
                dg[br] = pltpu.async_copy(ego_hbm.at[colv.at[j + 3]],
                                          rbufs[br], sgs[br])
        dsc[0].wait()
        dsc[1].wait()
        return carry

    for pi in range(2):
        p = pi * NS + s
        pltpu.sync_copy(cnt_hbm.at[p], cntb)
        cnt16 = cntb[...]
        nsup = jnp.where(c == 0, cnt16[0], cnt16[1])
        lax.fori_loop(0, nsup, lambda i, cr: seg_body(i, p, cr), 0)

    plsc.subcore_barrier()
    pltpu.sync_copy(acc.at[pl.ds(s * RPT, RPT)],
                    out_hbm.at[c, pl.ds(s * RPT, RPT)])


_spmm = pl.kernel(
    _spmm_body,
    out_type=jax.ShapeDtypeStruct((NC, RPC, D), jnp.float32),
    mesh=plsc.VectorSubcoreMesh(core_axis_name="c", subcore_axis_name="s"),
    compiler_params=pltpu.CompilerParams(use_tc_tiling_on_sc=False),
    scratch_types=[
        pltpu.VMEM((SUP, K), jnp.int32),
        pltpu.VMEM((SUP, K), jnp.int32),
        pltpu.VMEM((SUP, K), jnp.float32),
        pltpu.VMEM((16,), jnp.int32),
        pltpu.VMEM((K, D), jnp.float32),
        pltpu.VMEM((K, D), jnp.float32),
        pltpu.VMEM((K, D), jnp.float32),
        pltpu.VMEM((K, D), jnp.float32),
        pltpu.VMEM((K, D), jnp.float32),
        pltpu.VMEM_SHARED((RPC, D), jnp.float32),
        pltpu.SemaphoreType.DMA,
        pltpu.SemaphoreType.DMA,
        pltpu.SemaphoreType.DMA,
        pltpu.SemaphoreType.DMA,
        pltpu.SemaphoreType.DMA,
    ],
)


def _dense_body(side_ref, ego_ref, gwt_ref, gb_ref, bwt_ref, bb_ref,
                newe_ref, norm_ref):
    sd = side_ref[...]
    eg = ego_ref[...]
    sum_emb = jnp.dot(sd, gwt_ref[...], preferred_element_type=jnp.float32)
    sum_emb = sum_emb + gb_ref[...]
    sum_emb = jnp.where(sum_emb >= 0, sum_emb, 0.01 * sum_emb)
    bi = jnp.dot(eg * sd, bwt_ref[...], preferred_element_type=jnp.float32)
    bi = bi + bb_ref[...]
    bi = jnp.where(bi >= 0, bi, 0.01 * bi)
    new = sum_emb + bi
    newe_ref[...] = new
    nrm = jnp.sqrt(jnp.sum(new * new, axis=1, keepdims=True))
    norm_ref[...] = new / jnp.maximum(nrm, 1e-12)


_ROWS = NC * RPC
_BLK = 1024

_dense = pl.pallas_call(
    _dense_body,
    grid=(_ROWS // _BLK,),
    in_specs=[
        pl.BlockSpec((_BLK, D), lambda i: (i, 0)),
        pl.BlockSpec((_BLK, D), lambda i: (i, 0)),
        pl.BlockSpec((D, D), lambda i: (0, 0)),
        pl.BlockSpec((1, D), lambda i: (0, 0)),
        pl.BlockSpec((D, D), lambda i: (0, 0)),
        pl.BlockSpec((1, D), lambda i: (0, 0)),
    ],
    out_specs=[
        pl.BlockSpec((_BLK, D), lambda i: (i, 0)),
        pl.BlockSpec((_BLK, D), lambda i: (i, 0)),
    ],
    out_shape=[
        jax.ShapeDtypeStruct((_ROWS, D), jnp.float32),
        jax.ShapeDtypeStruct((_ROWS, D), jnp.float32),
    ],
)


def kernel(adj_indices, adj_values, build_item_graph, user_emb, item_emb,
           gc_w0, gc_b0, bi_w0, bi_b0, gc_w1, gc_b1, bi_w1, bi_b1):
    row = adj_indices[0].astype(jnp.int32)
    col = adj_indices[1].astype(jnp.int32)
    val = adj_values.astype(jnp.float32)

    col = col + PAD * (col >= HALF).astype(jnp.int32)
    npad = EPAD - E
    row = jnp.concatenate([row, jnp.full((npad,), N, jnp.int32)])
    col = jnp.concatenate([col, jnp.zeros((npad,), jnp.int32)])
    val = jnp.concatenate([val, jnp.zeros((npad,), jnp.float32)])

    col2, lr2, val2, cnt = _route(col, row, val)

    ego_p = jnp.zeros((NC, RPC, D), jnp.float32)
    ego_p = ego_p.at[0, :HALF].set(user_emb)
    ego_p = ego_p.at[1, :HALF].set(item_emb)
    zeros = jnp.zeros((RPT, D), jnp.float32)

    norms = []
    for (gw, gb, bw, bb) in ((gc_w0, gc_b0, bi_w0, bi_b0),
                             (gc_w1, gc_b1, bi_w1, bi_b1)):
        side_p = _spmm(ego_p.reshape(_ROWS, D), col2, lr2, val2, cnt, zeros)
        newe, norm = _dense(side_p.reshape(_ROWS, D), ego_p.reshape(_ROWS, D),
                            gw.T, gb.reshape(1, D), bw.T, bb.reshape(1, D))
        ego_p = newe.reshape(NC, RPC, D)
        norms.append(norm.reshape(NC, RPC, D))

    u_g = jnp.concatenate(
        [user_emb, norms[0][0, :HALF], norms[1][0, :HALF]], axis=1)
    i_g = jnp.concatenate(
        [item_emb, norms[0][1, :HALF], norms[1][1, :HALF]], axis=1)
    return (u_g, i_g)


# speedup vs baseline: 1.0946x; 1.0946x over previous
import jax
import jax.numpy as jnp
from jax import lax
from jax.experimental import pallas as pl
from jax.experimental.pallas import tpu as pltpu
from jax.experimental.pallas import tpu_sc as plsc

N_USERS = 25000
N_ITEMS = 25000
N = N_USERS + N_ITEMS
E = 800000
D = 64

NC = 2
NS = 16
NW = NC * NS
HALF = N // NC
RPC = 25088
RPT = RPC // NS
DUMP = HALF
PAD = RPC - HALF
K = 80
SUP = 8
KS = K * SUP
EPS = 25600
EPAD = NW * EPS
NSUP_IN = EPS // KS
CAPB = EPS // K


def _route_body(colf_hbm, rowf_hbm, valf_hbm,
                col2_hbm, lr2_hbm, val2_hbm, cnt_hbm,
                colv, rowv, valv, oc0, ol0, ov0, oc1, ol1, ov1, cntv,
                sgi0, sgi1, sgi2):
    c = lax.axis_index("c")
    s = lax.axis_index("s")
    w = c * NS + s
    e0 = w * EPS
    ocs = (oc0, oc1)
    ols = (ol0, ol1)
    ovs = (ov0, ov1)
    iota = lax.broadcasted_iota(jnp.int32, (16,), 0)

    def flush(cc, blk):
        for r in range(SUP):
            pltpu.sync_copy(ocs[cc].at[pl.ds(r * K, K)],
                            col2_hbm.at[cc, w, blk * SUP + r])
            pltpu.sync_copy(ols[cc].at[pl.ds(r * K, K)],
                            lr2_hbm.at[cc, w, blk * SUP + r])
            pltpu.sync_copy(ovs[cc].at[pl.ds(r * K, K)],
                            val2_hbm.at[cc, w, blk * SUP + r])

    def sup_body(i, carry):
        di0 = pltpu.async_copy(colf_hbm.at[pl.ds(e0 + i * KS, KS)], colv,
                               sgi0)
        di1 = pltpu.async_copy(rowf_hbm.at[pl.ds(e0 + i * KS, KS)], rowv,
                               sgi1)
        di2 = pltpu.async_copy(valf_hbm.at[pl.ds(e0 + i * KS, KS)], valv,
                               sgi2)
        di0.wait()
        di1.wait()
        di2.wait()

        def grp(g, carry2):
            ptrs = [carry2[0], carry2[2]]
            blks = [carry2[1], carry2[3]]
            sl = pl.ds(g * 16, 16)
            c16 = colv[sl]
            r16 = rowv[sl]
            v16 = valv[sl]
            for cc in range(NC):
                ptr = ptrs[cc]
                blk = blks[cc]
                lr = r16 - cc * HALF
                m = (lr >= 0) & (lr < HALF)
                cnt = plsc.all_reduce_population_count(m)[0]
                plsc.store_compressed(ocs[cc].at[pl.ds(ptr, 16)], c16, mask=m)
                plsc.store_compressed(ols[cc].at[pl.ds(ptr, 16)], lr, mask=m)
                plsc.store_compressed(ovs[cc].at[pl.ds(ptr, 16)], v16, mask=m)
                pnew = ptr + cnt
                full = pnew >= KS

                @pl.when(full)
                def _():
                    flush(cc, blk)
                    for t in range(K // 16):
                        dst = pl.ds(t * 16, 16)
                        ssl = pl.ds(KS + t * 16, 16)
                        ocs[cc][dst] = ocs[cc][ssl]
                        ols[cc][dst] = ols[cc][ssl]
                        ovs[cc][dst] = ovs[cc][ssl]

                ptrs[cc] = jnp.where(full, pnew - KS, pnew)
                blks[cc] = jnp.where(full, blk + 1, blk)
            return (ptrs[0], blks[0], ptrs[1], blks[1])

        return lax.fori_loop(0, KS // 16, grp, carry)

    z = jnp.int32(0)
    p0, b0, p1, b1 = lax.fori_loop(0, NSUP_IN, sup_body, (z, z, z, z))

    nsups = []
    for cc, (ptr, blk) in enumerate(((p0, b0), (p1, b1))):
        def pad_grp(t, carry2):
            ofs = ptr + t * 16

            @pl.when(ofs < KS)
            def _():
                ocs[cc][pl.ds(ofs, 16)] = iota * 0
                ols[cc][pl.ds(ofs, 16)] = iota * 0 + DUMP
                ovs[cc][pl.ds(ofs, 16)] = (iota * 0).astype(jnp.float32)

            return carry2

        lax.fori_loop(0, KS // 16, pad_grp, 0)

        @pl.when(ptr > 0)
        def _():
            flush(cc, blk)

        nsups.append(blk + (ptr > 0).astype(jnp.int32))

    cntv[...] = jnp.where(iota == 0, nsups[0],
                          jnp.where(iota == 1, nsups[1], 0))
    pltpu.sync_copy(cntv, cnt_hbm.at[w])


_route = pl.kernel(
    _route_body,
    out_type=[
        jax.ShapeDtypeStruct((NC, NW, CAPB, K), jnp.int32),
        jax.ShapeDtypeStruct((NC, NW, CAPB, K), jnp.int32),
        jax.ShapeDtypeStruct((NC, NW, CAPB, K), jnp.float32),
        jax.ShapeDtypeStruct((NW, 16), jnp.int32),
    ],
    mesh=plsc.VectorSubcoreMesh(core_axis_name="c", subcore_axis_name="s"),
    compiler_params=pltpu.CompilerParams(use_tc_tiling_on_sc=False,
                                         needs_layout_passes=False),
    scratch_types=[
        pltpu.VMEM((KS,), jnp.int32),
        pltpu.VMEM((KS,), jnp.int32),
        pltpu.VMEM((KS,), jnp.float32),
        pltpu.VMEM(((SUP + 1) * K,), jnp.int32),
        pltpu.VMEM(((SUP + 1) * K,), jnp.int32),
        pltpu.VMEM(((SUP + 1) * K,), jnp.float32),
        pltpu.VMEM(((SUP + 1) * K,), jnp.int32),
        pltpu.VMEM(((SUP + 1) * K,), jnp.int32),
        pltpu.VMEM(((SUP + 1) * K,), jnp.float32),
        pltpu.VMEM((16,), jnp.int32),
        pltpu.SemaphoreType.DMA,
        pltpu.SemaphoreType.DMA,
        pltpu.SemaphoreType.DMA,
    ],
)


def _spmm_body(ego_hbm, col2_hbm, lr2_hbm, val2_hbm, cnt_hbm, zeros_hbm,
               out_hbm,
               colv, lrm, vlm, cntb, rb0, rb1, rb2, sb0, sb1, acc,
               sg0, sg1, sg2, ss0, ss1):
    c = lax.axis_index("c")
    s = lax.axis_index("s")
    rbufs = (rb0, rb1, rb2)
    sbufs = (sb0, sb1)
    sgs = (sg0, sg1, sg2)
    sss = (ss0, ss1)

    pltpu.sync_copy(zeros_hbm, acc.at[pl.ds(s * RPT, RPT)])
    plsc.subcore_barrier()

    def mul_pass(rbuf, sbuf, j):
        @plsc.parallel_loop(0, K // 16, unroll=2)
        def group(g):
            v16 = vlm[j, pl.ds(g * 16, 16)]
            for e in range(16):
                k = g * 16 + e
                ve = jnp.take(v16, jnp.full((16,), e, jnp.int32))
                for d_ in range(D // 16):
                    sl = pl.ds(d_ * 16, 16)
                    sbuf[k, sl] = rbuf[k, sl] * ve

    def seg_body(sc_i, p, carry):
        base = pl.ds(sc_i * SUP, SUP)
        di0 = pltpu.async_copy(col2_hbm.at[c, p, base], colv, sg0)
        di1 = pltpu.async_copy(lr2_hbm.at[c, p, base], lrm, sg1)
        di2 = pltpu.async_copy(val2_hbm.at[c, p, base], vlm, sg2)
        di0.wait()
        di1.wait()
        di2.wait()

        dg = [None, None, None]
        dsc = [None, None]
        for jb in range(3):
            dg[jb] = pltpu.async_copy(ego_hbm.at[colv.at[jb]], rbufs[jb],
                                      sgs[jb])
        for j in range(SUP):
            br = j % 3
            bs = j % 2
            dg[br].wait()
            if j >= 2:
                dsc[bs].wait()
            mul_pass(rbufs[br], sbufs[bs], j)
            dsc[bs] = pltpu.async_copy(sbufs[bs], acc.at[lrm.at[j]],
                                       sss[bs], add=True)
            if j + 3 < SUP:
                dg[br] = pltpu.async_copy(ego_hbm.at[colv.at[j + 3]],
                                          rbufs[br], sgs[br])
        dsc[0].wait()
        dsc[1].wait()
        return carry

    for pi in range(2):
        p = pi * NS + s
        pltpu.sync_copy(cnt_hbm.at[p], cntb)
        cnt16 = cntb[...]
        nsup = jnp.where(c == 0, cnt16[0], cnt16[1])
        lax.fori_loop(0, nsup, lambda i, cr: seg_body(i, p, cr), 0)

    plsc.subcore_barrier()
    pltpu.sync_copy(acc.at[pl.ds(s * RPT, RPT)],
                    out_hbm.at[c, pl.ds(s * RPT, RPT)])


_spmm = pl.kernel(
    _spmm_body,
    out_type=jax.ShapeDtypeStruct((NC, RPC, D), jnp.float32),
    mesh=plsc.VectorSubcoreMesh(core_axis_name="c", subcore_axis_name="s"),
    compiler_params=pltpu.CompilerParams(use_tc_tiling_on_sc=False),
    scratch_types=[
        pltpu.VMEM((SUP, K), jnp.int32),
        pltpu.VMEM((SUP, K), jnp.int32),
        pltpu.VMEM((SUP, K), jnp.float32),
        pltpu.VMEM((16,), jnp.int32),
        pltpu.VMEM((K, D), jnp.float32),
        pltpu.VMEM((K, D), jnp.float32),
        pltpu.VMEM((K, D), jnp.float32),
        pltpu.VMEM((K, D), jnp.float32),
        pltpu.VMEM((K, D), jnp.float32),
        pltpu.VMEM_SHARED((RPC, D), jnp.float32),
        pltpu.SemaphoreType.DMA,
        pltpu.SemaphoreType.DMA,
        pltpu.SemaphoreType.DMA,
        pltpu.SemaphoreType.DMA,
        pltpu.SemaphoreType.DMA,
    ],
)


def _dense_body(side_ref, ego_ref, gwt_ref, gb_ref, bwt_ref, bb_ref,
                newe_ref, norm_ref):
    sd = side_ref[...]
    eg = ego_ref[...]
    sum_emb = jnp.dot(sd, gwt_ref[...], preferred_element_type=jnp.float32)
    sum_emb = sum_emb + gb_ref[...]
    sum_emb = jnp.where(sum_emb >= 0, sum_emb, 0.01 * sum_emb)
    bi = jnp.dot(eg * sd, bwt_ref[...], preferred_element_type=jnp.float32)
    bi = bi + bb_ref[...]
    bi = jnp.where(bi >= 0, bi, 0.01 * bi)
    new = sum_emb + bi
    newe_ref[...] = new
    nrm = jnp.sqrt(jnp.sum(new * new, axis=1, keepdims=True))
    norm_ref[...] = new / jnp.maximum(nrm, 1e-12)


_ROWS = NC * RPC
_BLK = 1024

_dense = pl.pallas_call(
    _dense_body,
    grid=(_ROWS // _BLK,),
    in_specs=[
        pl.BlockSpec((_BLK, D), lambda i: (i, 0)),
        pl.BlockSpec((_BLK, D), lambda i: (i, 0)),
        pl.BlockSpec((D, D), lambda i: (0, 0)),
        pl.BlockSpec((1, D), lambda i: (0, 0)),
        pl.BlockSpec((D, D), lambda i: (0, 0)),
        pl.BlockSpec((1, D), lambda i: (0, 0)),
    ],
    out_specs=[
        pl.BlockSpec((_BLK, D), lambda i: (i, 0)),
        pl.BlockSpec((_BLK, D), lambda i: (i, 0)),
    ],
    out_shape=[
        jax.ShapeDtypeStruct((_ROWS, D), jnp.float32),
        jax.ShapeDtypeStruct((_ROWS, D), jnp.float32),
    ],
)


def kernel(adj_indices, adj_values, build_item_graph, user_emb, item_emb,
           gc_w0, gc_b0, bi_w0, bi_b0, gc_w1, gc_b1, bi_w1, bi_b1):
    row = adj_indices[0].astype(jnp.int32)
    col = adj_indices[1].astype(jnp.int32)
    val = adj_values.astype(jnp.float32)

    col = col + PAD * (col >= HALF).astype(jnp.int32)
    npad = EPAD - E
    row = jnp.concatenate([row, jnp.full((npad,), N, jnp.int32)])
    col = jnp.concatenate([col, jnp.zeros((npad,), jnp.int32)])
    val = jnp.concatenate([val, jnp.zeros((npad,), jnp.float32)])

    col2, lr2, val2, cnt = _route(col, row, val)

    ego_p = jnp.zeros((NC, RPC, D), jnp.float32)
    ego_p = ego_p.at[0, :HALF].set(user_emb)
    ego_p = ego_p.at[1, :HALF].set(item_emb)
    zeros = jnp.zeros((RPT, D), jnp.float32)

    norms = []
    for (gw, gb, bw, bb) in ((gc_w0, gc_b0, bi_w0, bi_b0),
                             (gc_w1, gc_b1, bi_w1, bi_b1)):
        side_p = _spmm(ego_p.reshape(_ROWS, D), col2, lr2, val2, cnt, zeros)
        newe, norm = _dense(side_p.reshape(_ROWS, D), ego_p.reshape(_ROWS, D),
                            gw.T, gb.reshape(1, D), bw.T, bb.reshape(1, D))
        ego_p = newe.reshape(NC, RPC, D)
        norms.append(norm.reshape(NC, RPC, D))

    u_g = jnp.concatenate(
        [user_emb, norms[0][0, :HALF], norms[1][0, :HALF]], axis=1)
    i_g = jnp.concatenate(
        [item_emb, norms[0][1, :HALF], norms[1][1, :HALF]], axis=1)
    return (u_g, i_g)


# speedup vs baseline: 1.1419x; 1.0432x over previous
import jax
import jax.numpy as jnp
from jax import lax
from jax.experimental import pallas as pl
from jax.experimental.pallas import tpu as pltpu
from jax.experimental.pallas import tpu_sc as plsc

N_USERS = 25000
N_ITEMS = 25000
N = N_USERS + N_ITEMS
E = 800000
D = 64

NC = 2
NS = 16
NW = NC * NS
HALF = N // NC
RPC = 25088
RPT = RPC // NS
DUMP = HALF
PAD = RPC - HALF
K = 80
SUP = 8
KS = K * SUP
EPS = 25600
EPAD = NW * EPS
KSI = 1600
NSUP_IN = EPS // KSI
CAPB = EPS // K


def _route_body(colf_hbm, rowf_hbm, valf_hbm,
                col2_hbm, lr2_hbm, val2_hbm, cnt_hbm,
                colv, rowv, valv, oc0, ol0, ov0, oc1, ol1, ov1, cntv,
                sgi0, sgi1, sgi2):
    c = lax.axis_index("c")
    s = lax.axis_index("s")
    w = c * NS + s
    e0 = w * EPS
    ocs = (oc0, oc1)
    ols = (ol0, ol1)
    ovs = (ov0, ov1)
    iota = lax.broadcasted_iota(jnp.int32, (16,), 0)

    def flush(cc, blk):
        for r in range(SUP):
            pltpu.sync_copy(ocs[cc].at[pl.ds(r * K, K)],
                            col2_hbm.at[cc, w, blk * SUP + r])
            pltpu.sync_copy(ols[cc].at[pl.ds(r * K, K)],
                            lr2_hbm.at[cc, w, blk * SUP + r])
            pltpu.sync_copy(ovs[cc].at[pl.ds(r * K, K)],
                            val2_hbm.at[cc, w, blk * SUP + r])

    def sup_body(i, carry):
        di0 = pltpu.async_copy(colf_hbm.at[pl.ds(e0 + i * KSI, KSI)], colv,
                               sgi0)
        di1 = pltpu.async_copy(rowf_hbm.at[pl.ds(e0 + i * KSI, KSI)], rowv,
                               sgi1)
        di2 = pltpu.async_copy(valf_hbm.at[pl.ds(e0 + i * KSI, KSI)], valv,
                               sgi2)
        di0.wait()
        di1.wait()
        di2.wait()

        def grp(g, carry2):
            ptrs = [carry2[0], carry2[2]]
            blks = [carry2[1], carry2[3]]
            sl = pl.ds(g * 16, 16)
            c16 = colv[sl]
            r16 = rowv[sl]
            v16 = valv[sl]
            for cc in range(NC):
                ptr = ptrs[cc]
                blk = blks[cc]
                lr = r16 - cc * HALF
                m = (lr >= 0) & (lr < HALF)
                cnt = plsc.all_reduce_population_count(m)[0]
                plsc.store_compressed(ocs[cc].at[pl.ds(ptr, 16)], c16, mask=m)
                plsc.store_compressed(ols[cc].at[pl.ds(ptr, 16)], lr, mask=m)
                plsc.store_compressed(ovs[cc].at[pl.ds(ptr, 16)], v16, mask=m)
                pnew = ptr + cnt
                full = pnew >= KS

                @pl.when(full)
                def _():
                    flush(cc, blk)
                    for t in range(K // 16):
                        dst = pl.ds(t * 16, 16)
                        ssl = pl.ds(KS + t * 16, 16)
                        ocs[cc][dst] = ocs[cc][ssl]
                        ols[cc][dst] = ols[cc][ssl]
                        ovs[cc][dst] = ovs[cc][ssl]

                ptrs[cc] = jnp.where(full, pnew - KS, pnew)
                blks[cc] = jnp.where(full, blk + 1, blk)
            return (ptrs[0], blks[0], ptrs[1], blks[1])

        return lax.fori_loop(0, KSI // 16, grp, carry)

    z = jnp.int32(0)
    p0, b0, p1, b1 = lax.fori_loop(0, NSUP_IN, sup_body, (z, z, z, z))

    nsups = []
    for cc, (ptr, blk) in enumerate(((p0, b0), (p1, b1))):
        def pad_grp(t, carry2):
            ofs = ptr + t * 16

            @pl.when(ofs < KS)
            def _():
                ocs[cc][pl.ds(ofs, 16)] = iota * 0
                ols[cc][pl.ds(ofs, 16)] = iota * 0 + DUMP
                ovs[cc][pl.ds(ofs, 16)] = (iota * 0).astype(jnp.float32)

            return carry2

        lax.fori_loop(0, KS // 16, pad_grp, 0)

        @pl.when(ptr > 0)
        def _():
            flush(cc, blk)

        nsups.append(blk + (ptr > 0).astype(jnp.int32))

    cntv[...] = jnp.where(iota == 0, nsups[0],
                          jnp.where(iota == 1, nsups[1], 0))
    pltpu.sync_copy(cntv, cnt_hbm.at[w])


_route = pl.kernel(
    _route_body,
    out_type=[
        jax.ShapeDtypeStruct((NC, NW, CAPB, K), jnp.int32),
        jax.ShapeDtypeStruct((NC, NW, CAPB, K), jnp.int32),
        jax.ShapeDtypeStruct((NC, NW, CAPB, K), jnp.float32),
        jax.ShapeDtypeStruct((NW, 16), jnp.int32),
    ],
    mesh=plsc.VectorSubcoreMesh(core_axis_name="c", subcore_axis_name="s"),
    compiler_params=pltpu.CompilerParams(use_tc_tiling_on_sc=False,
                                         needs_layout_passes=False),
    scratch_types=[
        pltpu.VMEM((KSI,), jnp.int32),
        pltpu.VMEM((KSI,), jnp.int32),
        pltpu.VMEM((KSI,), jnp.float32),
        pltpu.VMEM(((SUP + 1) * K,), jnp.int32),
        pltpu.VMEM(((SUP + 1) * K,), jnp.int32),
        pltpu.VMEM(((SUP + 1) * K,), jnp.float32),
        pltpu.VMEM(((SUP + 1) * K,), jnp.int32),
        pltpu.VMEM(((SUP + 1) * K,), jnp.int32),
        pltpu.VMEM(((SUP + 1) * K,), jnp.float32),
        pltpu.VMEM((16,), jnp.int32),
        pltpu.SemaphoreType.DMA,
        pltpu.SemaphoreType.DMA,
        pltpu.SemaphoreType.DMA,
    ],
)


def _spmm_body(ego_hbm, col2_hbm, lr2_hbm, val2_hbm, cnt_hbm, zeros_hbm,
               out_hbm,
               colv, lrm, vlm, cntb, rb0, rb1, rb2, sb0, sb1, acc,
               sg0, sg1, sg2, ss0, ss1):
    c = lax.axis_index("c")
    s = lax.axis_index("s")
    rbufs = (rb0, rb1, rb2)
    sbufs = (sb0, sb1)
    sgs = (sg0, sg1, sg2)
    sss = (ss0, ss1)

    pltpu.sync_copy(zeros_hbm, acc.at[pl.ds(s * RPT, RPT)])
    plsc.subcore_barrier()

    def mul_pass(rbuf, sbuf, j):
        @plsc.parallel_loop(0, K // 16, unroll=2)
        def group(g):
            v16 = vlm[j, pl.ds(g * 16, 16)]
            for e in range(16):
                k = g * 16 + e
                ve = jnp.take(v16, jnp.full((16,), e, jnp.int32))
                for d_ in range(D // 16):
                    sl = pl.ds(d_ * 16, 16)
                    sbuf[k, sl] = rbuf[k, sl] * ve

    def seg_body(sc_i, p, carry):
        base = pl.ds(sc_i * SUP, SUP)
        di0 = pltpu.async_copy(col2_hbm.at[c, p, base], colv, sg0)
        di1 = pltpu.async_copy(lr2_hbm.at[c, p, base], lrm, sg1)
        di2 = pltpu.async_copy(val2_hbm.at[c, p, base], vlm, sg2)
        di0.wait()
        di1.wait()
        di2.wait()

        dg = [None, None, None]
        dsc = [None, None]
        for jb in range(3):
            dg[jb] = pltpu.async_copy(ego_hbm.at[colv.at[jb]], rbufs[jb],
                                      sgs[jb])
        for j in range(SUP):
            br = j % 3
            bs = j % 2
            dg[br].wait()
            if j >= 2:
                dsc[bs].wait()
            mul_pass(rbufs[br], sbufs[bs], j)
            dsc[bs] = pltpu.async_copy(sbufs[bs], acc.at[lrm.at[j]],
                                       sss[bs], add=True)
            if j + 3 < SUP:
                dg[br] = pltpu.async_copy(ego_hbm.at[colv.at[j + 3]],
                                          rbufs[br], sgs[br])
        dsc[0].wait()
        dsc[1].wait()
        return carry

    for pi in range(2):
        p = pi * NS + s
        pltpu.sync_copy(cnt_hbm.at[p], cntb)
        cnt16 = cntb[...]
        nsup = jnp.where(c == 0, cnt16[0], cnt16[1])
        lax.fori_loop(0, nsup, lambda i, cr: seg_body(i, p, cr), 0)

    plsc.subcore_barrier()
    pltpu.sync_copy(acc.at[pl.ds(s * RPT, RPT)],
                    out_hbm.at[c, pl.ds(s * RPT, RPT)])


_spmm = pl.kernel(
    _spmm_body,
    out_type=jax.ShapeDtypeStruct((NC, RPC, D), jnp.float32),
    mesh=plsc.VectorSubcoreMesh(core_axis_name="c", subcore_axis_name="s"),
    compiler_params=pltpu.CompilerParams(use_tc_tiling_on_sc=False),
    scratch_types=[
        pltpu.VMEM((SUP, K), jnp.int32),
        pltpu.VMEM((SUP, K), jnp.int32),
        pltpu.VMEM((SUP, K), jnp.float32),
        pltpu.VMEM((16,), jnp.int32),
        pltpu.VMEM((K, D), jnp.float32),
        pltpu.VMEM((K, D), jnp.float32),
        pltpu.VMEM((K, D), jnp.float32),
        pltpu.VMEM((K, D), jnp.float32),
        pltpu.VMEM((K, D), jnp.float32),
        pltpu.VMEM_SHARED((RPC, D), jnp.float32),
        pltpu.SemaphoreType.DMA,
        pltpu.SemaphoreType.DMA,
        pltpu.SemaphoreType.DMA,
        pltpu.SemaphoreType.DMA,
        pltpu.SemaphoreType.DMA,
    ],
)


def _dense_body(side_ref, ego_ref, gwt_ref, gb_ref, bwt_ref, bb_ref,
                newe_ref, norm_ref):
    sd = side_ref[...]
    eg = ego_ref[...]
    sum_emb = jnp.dot(sd, gwt_ref[...], preferred_element_type=jnp.float32)
    sum_emb = sum_emb + gb_ref[...]
    sum_emb = jnp.where(sum_emb >= 0, sum_emb, 0.01 * sum_emb)
    bi = jnp.dot(eg * sd, bwt_ref[...], preferred_element_type=jnp.float32)
    bi = bi + bb_ref[...]
    bi = jnp.where(bi >= 0, bi, 0.01 * bi)
    new = sum_emb + bi
    newe_ref[...] = new
    nrm = jnp.sqrt(jnp.sum(new * new, axis=1, keepdims=True))
    norm_ref[...] = new / jnp.maximum(nrm, 1e-12)


_ROWS = NC * RPC
_BLK = 3584

_dense = pl.pallas_call(
    _dense_body,
    grid=(_ROWS // _BLK,),
    in_specs=[
        pl.BlockSpec((_BLK, D), lambda i: (i, 0)),
        pl.BlockSpec((_BLK, D), lambda i: (i, 0)),
        pl.BlockSpec((D, D), lambda i: (0, 0)),
        pl.BlockSpec((1, D), lambda i: (0, 0)),
        pl.BlockSpec((D, D), lambda i: (0, 0)),
        pl.BlockSpec((1, D), lambda i: (0, 0)),
    ],
    out_specs=[
        pl.BlockSpec((_BLK, D), lambda i: (i, 0)),
        pl.BlockSpec((_BLK, D), lambda i: (i, 0)),
    ],
    out_shape=[
        jax.ShapeDtypeStruct((_ROWS, D), jnp.float32),
        jax.ShapeDtypeStruct((_ROWS, D), jnp.float32),
    ],
)


def kernel(adj_indices, adj_values, build_item_graph, user_emb, item_emb,
           gc_w0, gc_b0, bi_w0, bi_b0, gc_w1, gc_b1, bi_w1, bi_b1):
    row = adj_indices[0].astype(jnp.int32)
    col = adj_indices[1].astype(jnp.int32)
    val = adj_values.astype(jnp.float32)

    col = col + PAD * (col >= HALF).astype(jnp.int32)
    npad = EPAD - E
    row = jnp.concatenate([row, jnp.full((npad,), N, jnp.int32)])
    col = jnp.concatenate([col, jnp.zeros((npad,), jnp.int32)])
    val = jnp.concatenate([val, jnp.zeros((npad,), jnp.float32)])

    col2, lr2, val2, cnt = _route(col, row, val)

    ego_p = jnp.zeros((NC, RPC, D), jnp.float32)
    ego_p = ego_p.at[0, :HALF].set(user_emb)
    ego_p = ego_p.at[1, :HALF].set(item_emb)
    zeros = jnp.zeros((RPT, D), jnp.float32)

    norms = []
    for (gw, gb, bw, bb) in ((gc_w0, gc_b0, bi_w0, bi_b0),
                             (gc_w1, gc_b1, bi_w1, bi_b1)):
        side_p = _spmm(ego_p.reshape(_ROWS, D), col2, lr2, val2, cnt, zeros)
        newe, norm = _dense(side_p.reshape(_ROWS, D), ego_p.reshape(_ROWS, D),
                            gw.T, gb.reshape(1, D), bw.T, bb.reshape(1, D))
        ego_p = newe.reshape(NC, RPC, D)
        norms.append(norm.reshape(NC, RPC, D))

    u_g = jnp.concatenate(
        [user_emb, norms[0][0, :HALF], norms[1][0, :HALF]], axis=1)
    i_g = jnp.concatenate(
        [item_emb, norms[0][1, :HALF], norms[1][1, :HALF]], axis=1)
    return (u_g, i_g)
